# baseline (device time: 85918 ns/iter reference)
import jax
import jax.numpy as jnp
from jax import lax
from jax.experimental import pallas as pl
from jax.experimental.pallas import tpu as pltpu

N_DEV = 4
N_HOPS = 2 * (N_DEV - 1)
SUB = 4
N_STAGE = 4


def kernel(x, w_mat):
    m, k_per = x.shape
    _, n = w_mat.shape
    rows = m // N_DEV
    n2 = n // 2
    nsub = n2 // SUB

    f32 = jnp.float32
    bf16 = jnp.bfloat16

    def body(
        x_ref,
        w_ref,
        out_ref,
        acc_ref,
        w_bf,
        sbuf_r,
        sbuf_l,
        rbuf_r,
        rbuf_l,
        ssem_r,
        rsem_r,
        ssem_l,
        rsem_l,
        osem_r,
        osem_l,
    ):
        my = lax.axis_index("i")
        right = lax.rem(my + 1, N_DEV)
        left = lax.rem(my + 3, N_DEV)

        def idx(d):
            return lax.rem(my + d + N_DEV, N_DEV)

        def gemm_block(c):
            acc_ref[pl.ds(c * rows, rows), :] = jnp.dot(
                x_ref[pl.ds(c * rows, rows), :].astype(bf16),
                w_bf[:, :],
                preferred_element_type=f32,
            ).astype(bf16)

        def acc_sub(c, half, s):
            return acc_ref[pl.ds(c * rows, rows), pl.ds(half * n2 + s * nsub, nsub)]

        R = (rbuf_r, ssem_r, rsem_r, right)
        L = (rbuf_l, ssem_l, rsem_l, left)

        def mk(dir_, src, h, s):
            rbuf, ssem, rsem, dev = dir_
            slot = h * SUB + s
            return pltpu.make_async_remote_copy(
                src_ref=src,
                dst_ref=rbuf.at[slot],
                send_sem=ssem.at[slot],
                recv_sem=rsem.at[slot],
                device_id=(dev,),
                device_id_type=pl.DeviceIdType.MESH,
            )

        ops = {}
        stores = []

        def start(key, dir_, src, h, s):
            op = mk(dir_, src, h, s)
            op.start()
            ops[key, h, s] = op

        def store_out(src, row_c, half, k, s, osem):
            cp = pltpu.make_async_copy(
                src,
                out_ref.at[
                    pl.ds(row_c * rows, rows),
                    pl.ds(half * n2 + s * nsub, nsub),
                ],
                osem.at[k * SUB + s],
            )
            cp.start()
            stores.append(cp)

        barrier = pltpu.get_barrier_semaphore()
        pl.semaphore_signal(
            barrier, inc=1, device_id=(left,),
            device_id_type=pl.DeviceIdType.MESH,
        )
        pl.semaphore_signal(
            barrier, inc=1, device_id=(right,),
            device_id_type=pl.DeviceIdType.MESH,
        )
        pl.semaphore_wait(barrier, 2)

        w_bf[:, :] = w_ref[:, :].astype(bf16)
        x_own = x_ref[pl.ds(my * rows, rows), :].astype(bf16)
        for s in range(SUB):
            sbuf_r[s, :, :] = jnp.dot(
                x_own, w_bf[:, pl.ds(s * nsub, nsub)],
                preferred_element_type=f32,
            ).astype(bf16)
            start("R", R, sbuf_r.at[s], 0, s)
            sbuf_l[s, :, :] = jnp.dot(
                x_own, w_bf[:, pl.ds(n2 + s * nsub, nsub)],
                preferred_element_type=f32,
            ).astype(bf16)
            start("L", L, sbuf_l.at[s], 0, s)
        gemm_block(idx(-1))
        gemm_block(idx(1))
        gemm_block(idx(2))

        for h, cR, cL in ((1, idx(-1), idx(1)), (2, idx(2), idx(2))):
            for s in range(SUB):
                slot = h * SUB + s
                ops["R", h - 1, s].wait_recv()
                sbuf_r[slot, :, :] = (
                    rbuf_r[(h - 1) * SUB + s].astype(f32)
                    + acc_sub(cR, 0, s).astype(f32)
                ).astype(bf16)
                start("R", R, sbuf_r.at[slot], h, s)
                ops["L", h - 1, s].wait_recv()
                sbuf_l[slot, :, :] = (
                    rbuf_l[(h - 1) * SUB + s].astype(f32)
                    + acc_sub(cL, 1, s).astype(f32)
                ).astype(bf16)
                start("L", L, sbuf_l.at[slot], h, s)

        for s in range(SUB):
            slot = 3 * SUB + s
            ops["R", 2, s].wait_recv()
            y = rbuf_r[2 * SUB + s].astype(f32) + acc_sub(idx(1), 0, s).astype(f32)
            sbuf_r[slot, :, :] = (y * (1.0 / (1.0 + jnp.exp(-y)))).astype(bf16)
            start("R", R, sbuf_r.at[slot], 3, s)
            store_out(sbuf_r.at[slot], idx(1), 0, 0, s, osem_r)

            ops["L", 2, s].wait_recv()
            y = rbuf_l[2 * SUB + s].astype(f32) + acc_sub(idx(-1), 1, s).astype(f32)
            sbuf_l[slot, :, :] = (y * (1.0 / (1.0 + jnp.exp(-y)))).astype(bf16)
            start("L", L, sbuf_l.at[slot], 3, s)
            store_out(sbuf_l.at[slot], idx(-1), 1, 0, s, osem_l)

        for h, dR, dL in ((4, 0, 0), (5, -1, 1)):
            for s in range(SUB):
                slot = (h - 1) * SUB + s
                ops["R", h - 1, s].wait_recv()
                start("R", R, rbuf_r.at[slot], h, s)
                store_out(rbuf_r.at[slot], idx(dR), 0, h - 3, s, osem_r)
                ops["L", h - 1, s].wait_recv()
                start("L", L, rbuf_l.at[slot], h, s)
                store_out(rbuf_l.at[slot], idx(dL), 1, h - 3, s, osem_l)

        for s in range(SUB):
            slot = 5 * SUB + s
            ops["R", 5, s].wait_recv()
            store_out(rbuf_r.at[slot], idx(-2), 0, 3, s, osem_r)
            ops["L", 5, s].wait_recv()
            store_out(rbuf_l.at[slot], idx(2), 1, 3, s, osem_l)
        for cp in stores:
            cp.wait()
        for h in range(N_HOPS):
            for s in range(SUB):
                ops["R", h, s].wait_send()
                ops["L", h, s].wait_send()

    return pl.pallas_call(
        body,
        out_shape=jax.ShapeDtypeStruct((m, n), bf16),
        in_specs=[
            pl.BlockSpec(memory_space=pltpu.VMEM),
            pl.BlockSpec(memory_space=pltpu.VMEM),
        ],
        out_specs=pl.BlockSpec(memory_space=pltpu.MemorySpace.HBM),
        scratch_shapes=[
            pltpu.VMEM((m, n), bf16),
            pltpu.VMEM((k_per, n), bf16),
            pltpu.VMEM((N_STAGE * SUB, rows, nsub), bf16),
            pltpu.VMEM((N_STAGE * SUB, rows, nsub), bf16),
            pltpu.VMEM((N_HOPS * SUB, rows, nsub), bf16),
            pltpu.VMEM((N_HOPS * SUB, rows, nsub), bf16),
            pltpu.SemaphoreType.DMA((N_HOPS * SUB,)),
            pltpu.SemaphoreType.DMA((N_HOPS * SUB,)),
            pltpu.SemaphoreType.DMA((N_HOPS * SUB,)),
            pltpu.SemaphoreType.DMA((N_HOPS * SUB,)),
            pltpu.SemaphoreType.DMA((4 * SUB,)),
            pltpu.SemaphoreType.DMA((4 * SUB,)),
        ],
        compiler_params=pltpu.CompilerParams(
            vmem_limit_bytes=100 * 1024 * 1024,
            collective_id=0,
        ),
    )(x, w_mat)
